# Initial kernel scaffold; baseline (speedup 1.0000x reference)
import functools

import jax
import jax.numpy as jnp
from jax import lax
from jax.experimental import pallas as pl
from jax.experimental.pallas import tpu as pltpu
from jax.experimental.pallas import tpu_sc as plsc

N = 10000
E = 320000
D = 128
R = 8

NC = 2
NS = 16
NW = NC * NS
LANES = 16

WROWS = 80
ROWS = NW * WROWS
EP = ROWS * 128
GB = 8
NGROUPS = WROWS // GB

SEGP = 80128
NP = 10128
CNT_PART = SEGP // NS
ACC_PART = NP // NS

_mesh = plsc.VectorSubcoreMesh(core_axis_name="c", subcore_axis_name="s",
                               num_cores=NC, num_subcores=NS)


def _y_body(x_ref, w_ref, y_ref):
    y_ref[...] = jnp.dot(x_ref[...], w_ref[0],
                         preferred_element_type=jnp.float32)


def _build_y(x, weight):
    bn = 1000
    return pl.pallas_call(
        _y_body,
        grid=(R, N // bn),
        in_specs=[
            pl.BlockSpec((bn, D), lambda r, i: (i, 0)),
            pl.BlockSpec((1, D, D), lambda r, i: (r, 0, 0)),
        ],
        out_specs=pl.BlockSpec((bn, D), lambda r, i: (r * (N // bn) + i, 0)),
        out_shape=jax.ShapeDtypeStruct((R * N, D), jnp.float32),
    )(x, weight)


def _count_body(dst_hbm, et_hbm, cnt_hbm, cnt_sh, dstv, etv, segv, ones):
    cidx = lax.axis_index("c")
    sid = lax.axis_index("s")
    wid = sid * NC + cidx

    @pl.loop(0, 128)
    def _z(i):
        ones[i, :] = jnp.zeros((LANES,), jnp.float32)

    zbase = sid * CNT_PART

    @pl.loop(0, CNT_PART // 128)
    def _zc(i):
        pltpu.sync_copy(ones, cnt_sh.at[pl.ds(zbase + i * 128, 128)])

    pltpu.sync_copy(ones.at[pl.ds(0, CNT_PART % 128)],
                    cnt_sh.at[pl.ds(zbase + (CNT_PART // 128) * 128,
                                    CNT_PART % 128)])

    @pl.loop(0, 128)
    def _o(i):
        ones[i, :] = jnp.ones((LANES,), jnp.float32)

    plsc.subcore_barrier()

    @pl.loop(0, NGROUPS)
    def _grp(g):
        rb = wid * WROWS + g * GB
        pltpu.sync_copy(dst_hbm.at[pl.ds(rb, GB)], dstv)
        pltpu.sync_copy(et_hbm.at[pl.ds(rb, GB)], etv)
        for j in range(GB):
            for l in range(8):
                sl = pl.ds(l * LANES, LANES)
                segv[j, sl] = dstv[j, sl] * R + etv[j, sl]
        for j in range(GB):
            pltpu.sync_copy(ones, cnt_sh.at[segv.at[j]], add=True)

    plsc.subcore_barrier()
    pltpu.sync_copy(cnt_sh.at[pl.ds(sid * CNT_PART, CNT_PART)],
                    cnt_hbm.at[cidx, pl.ds(sid * CNT_PART, CNT_PART)])


def _count_edges(dst2d, et2d):
    k = pl.kernel(
        _count_body,
        out_type=jax.ShapeDtypeStruct((NC, SEGP, LANES), jnp.float32),
        mesh=_mesh,
        scratch_types=[
            pltpu.VMEM_SHARED((SEGP, LANES), jnp.float32),
            pltpu.VMEM((GB, 128), jnp.int32),
            pltpu.VMEM((GB, 128), jnp.int32),
            pltpu.VMEM((GB, 128), jnp.int32),
            pltpu.VMEM((128, LANES), jnp.float32),
        ],
    )
    return k(dst2d, et2d)


def _scale_body(cnt_ref, scale_ref):
    c = jnp.max(cnt_ref[0], axis=-1) + jnp.max(cnt_ref[1], axis=-1)
    scale_ref[...] = 1.0 / jnp.clip(c, 1.0, None)


def _build_scale(cnt):
    b = 128
    return pl.pallas_call(
        _scale_body,
        grid=(SEGP // b,),
        in_specs=[pl.BlockSpec((NC, b, LANES), lambda i: (0, i, 0))],
        out_specs=pl.BlockSpec((b,), lambda i: (i,)),
        out_shape=jax.ShapeDtypeStruct((SEGP,), jnp.float32),
    )(cnt)


def _agg_body(src_hbm, et_hbm, dst_hbm, y_hbm, scale_hbm, out_hbm,
              acc_sh, stab, rows, srcv, etv, dstv, gidxv, scalev):
    cidx = lax.axis_index("c")
    sid = lax.axis_index("s")
    wid = sid * NC + cidx

    @pl.loop(0, 128)
    def _z(i):
        for l in range(8):
            rows[i, pl.ds(l * LANES, LANES)] = jnp.zeros((LANES,), jnp.float32)

    abase = sid * ACC_PART

    @pl.loop(0, ACC_PART // 128)
    def _za(i):
        pltpu.sync_copy(rows, acc_sh.at[pl.ds(abase + i * 128, 128)])

    pltpu.sync_copy(rows.at[pl.ds(0, ACC_PART % 128)],
                    acc_sh.at[pl.ds(abase + (ACC_PART // 128) * 128,
                                    ACC_PART % 128)])

    pltpu.sync_copy(scale_hbm, stab)

    plsc.subcore_barrier()

    @pl.loop(0, NGROUPS)
    def _grp(g):
        rb = wid * WROWS + g * GB
        pltpu.sync_copy(src_hbm.at[pl.ds(rb, GB)], srcv)
        pltpu.sync_copy(et_hbm.at[pl.ds(rb, GB)], etv)
        pltpu.sync_copy(dst_hbm.at[pl.ds(rb, GB)], dstv)
        for j in range(GB):
            for l in range(8):
                sl = pl.ds(l * LANES, LANES)
                et16 = etv[j, sl]
                gidxv[j, sl] = et16 * N + srcv[j, sl]
                scalev[j, sl] = plsc.load_gather(
                    stab, [dstv[j, sl] * R + et16])
        for j in range(GB):
            pltpu.sync_copy(y_hbm.at[gidxv.at[j]], rows)

            @pl.loop(0, 128)
            def _sc(kk):
                s = scalev[j, kk]
                for l in range(8):
                    sl = pl.ds(l * LANES, LANES)
                    rows[kk, sl] = rows[kk, sl] * s

            pltpu.sync_copy(rows, acc_sh.at[dstv.at[j]], add=True)

    plsc.subcore_barrier()
    pltpu.sync_copy(acc_sh.at[pl.ds(sid * ACC_PART, ACC_PART)],
                    out_hbm.at[cidx, pl.ds(sid * ACC_PART, ACC_PART)])


def _aggregate(src2d, et2d, dst2d, y, scale):
    k = pl.kernel(
        _agg_body,
        out_type=jax.ShapeDtypeStruct((NC, NP, D), jnp.float32),
        mesh=_mesh,
        scratch_types=[
            pltpu.VMEM_SHARED((NP, D), jnp.float32),
            pltpu.VMEM((SEGP,), jnp.float32),
            pltpu.VMEM((128, D), jnp.float32),
            pltpu.VMEM((GB, 128), jnp.int32),
            pltpu.VMEM((GB, 128), jnp.int32),
            pltpu.VMEM((GB, 128), jnp.int32),
            pltpu.VMEM((GB, 128), jnp.int32),
            pltpu.VMEM((GB, 128), jnp.float32),
        ],
    )
    return k(src2d, et2d, dst2d, y, scale)


def _final_body(acc_ref, x_ref, root_ref, bias_ref, out_ref):
    out_ref[...] = (acc_ref[0] + acc_ref[1]
                    + jnp.dot(x_ref[...], root_ref[...],
                              preferred_element_type=jnp.float32)
                    + bias_ref[...])


def _finalize(acc, x, root, bias):
    b = 500
    return pl.pallas_call(
        _final_body,
        grid=(N // b,),
        in_specs=[
            pl.BlockSpec((NC, b, D), lambda i: (0, i, 0)),
            pl.BlockSpec((b, D), lambda i: (i, 0)),
            pl.BlockSpec((D, D), lambda i: (0, 0)),
            pl.BlockSpec((1, D), lambda i: (0, 0)),
        ],
        out_specs=pl.BlockSpec((b, D), lambda i: (i, 0)),
        out_shape=jax.ShapeDtypeStruct((N, D), jnp.float32),
    )(acc[:, :N], x, root, bias.reshape(1, D))


def kernel(x, edge_index, edge_type, weight, root, bias):
    src = edge_index[0].astype(jnp.int32)
    dst = edge_index[1].astype(jnp.int32)
    et = edge_type.astype(jnp.int32)

    npad = EP - E
    src2d = jnp.concatenate(
        [src, jnp.zeros((npad,), jnp.int32)]).reshape(ROWS, 128)
    dst2d = jnp.concatenate(
        [dst, jnp.full((npad,), N, jnp.int32)]).reshape(ROWS, 128)
    et2d = jnp.concatenate(
        [et, jnp.zeros((npad,), jnp.int32)]).reshape(ROWS, 128)

    y = _build_y(x, weight)
    cnt = _count_edges(dst2d, et2d)
    scale = _build_scale(cnt)
    acc = _aggregate(src2d, et2d, dst2d, y, scale)
    return _finalize(acc, x, root, bias)


# speedup vs baseline: 2.9967x; 2.9967x over previous
import dataclasses
import functools

import jax
import jax.numpy as jnp
from jax import lax
from jax.experimental import pallas as pl
from jax.experimental.pallas import tpu as pltpu
from jax.experimental.pallas import tpu_sc as plsc

N = 10000
E = 320000
D = 128
R = 8

NC = 2
NS = 16
NW = NC * NS
LANES = 16

WROWS = 80
ROWS = NW * WROWS
EP = ROWS * 128
GB = 8
NGROUPS = WROWS // GB

SEGP = 81920
CROWS = SEGP // 128
CNT_PART = CROWS // NS
NP = 10112
ACC_PART = NP // NS
OH = 32

_mesh = plsc.VectorSubcoreMesh(core_axis_name="c", subcore_axis_name="s",
                               num_cores=NC, num_subcores=NS)

_sc_params = pltpu.CompilerParams()
if "needs_layout_passes" in pltpu.CompilerParams.__dataclass_fields__:
    _sc_params = dataclasses.replace(_sc_params, needs_layout_passes=False)


def _y_body(x_ref, w_ref, y_ref):
    y_ref[...] = jnp.dot(x_ref[...], w_ref[0],
                         preferred_element_type=jnp.float32)


def _build_y(x, weight):
    bn = 1000
    return pl.pallas_call(
        _y_body,
        grid=(R, N // bn),
        in_specs=[
            pl.BlockSpec((bn, D), lambda r, i: (i, 0)),
            pl.BlockSpec((1, D, D), lambda r, i: (r, 0, 0)),
        ],
        out_specs=pl.BlockSpec((bn, D), lambda r, i: (r * (N // bn) + i, 0)),
        out_shape=jax.ShapeDtypeStruct((R * N, D), jnp.float32),
    )(x, weight)


def _count_body(dst_hbm, et_hbm, cnt_hbm, cnt_sh, dstv, etv, rowv, grpv,
                lanev, onehot):
    cidx = lax.axis_index("c")
    sid = lax.axis_index("s")
    wid = sid * NC + cidx
    it = lax.iota(jnp.int32, LANES)

    @pl.loop(0, OH)
    def _z(i):
        rk = onehot.at[i]
        for l in range(8):
            rk[pl.ds(l * LANES, LANES)] = jnp.zeros((LANES,), jnp.float32)

    zbase = sid * CNT_PART
    pltpu.sync_copy(onehot, cnt_sh.at[pl.ds(zbase, OH)])
    pltpu.sync_copy(onehot.at[pl.ds(0, CNT_PART - OH)],
                    cnt_sh.at[pl.ds(zbase + OH, CNT_PART - OH)])

    plsc.subcore_barrier()

    @pl.loop(0, NGROUPS)
    def _grp(g):
        rb = wid * WROWS + g * GB
        pltpu.sync_copy(dst_hbm.at[pl.ds(rb, GB)], dstv)
        pltpu.sync_copy(et_hbm.at[pl.ds(rb, GB)], etv)

        @pl.loop(0, GB)
        def _idx(j):
            rd, re = dstv.at[j], etv.at[j]
            rg, rl = grpv.at[j], lanev.at[j]
            for l in range(8):
                sl = pl.ds(l * LANES, LANES)
                seg16 = rd[sl] * R + re[sl]
                rowv.at[j * 4 + l // 2][pl.ds((l % 2) * LANES, LANES)] = (
                    jax.lax.shift_right_logical(seg16, 7))
                rg[sl] = jax.lax.bitwise_and(
                    jax.lax.shift_right_logical(seg16, 4), 7)
                rl[sl] = jax.lax.bitwise_and(seg16, 15)

        @pl.loop(0, GB * 4)
        def _sub(q):
            j = lax.div(q, 4)
            base = lax.rem(q, 4) * OH
            rg, rl = grpv.at[j], lanev.at[j]
            gvec = [rg[pl.ds(base, LANES)], rg[pl.ds(base + LANES, LANES)]]
            lvec = [rl[pl.ds(base, LANES)], rl[pl.ds(base + LANES, LANES)]]
            gs = []
            for i in range(OH):
                gi = gvec[i // LANES][i % LANES]
                li = lvec[i // LANES][i % LANES]
                gs.append(gi)
                onehot.at[i][pl.ds(gi * LANES, LANES)] = jnp.where(
                    it == li, 1.0, 0.0).astype(jnp.float32)
            pltpu.sync_copy(onehot, cnt_sh.at[rowv.at[q]], add=True)
            for i in range(OH):
                onehot.at[i][pl.ds(gs[i] * LANES, LANES)] = jnp.zeros(
                    (LANES,), jnp.float32)

    plsc.subcore_barrier()
    pltpu.sync_copy(cnt_sh.at[pl.ds(sid * CNT_PART, CNT_PART)],
                    cnt_hbm.at[cidx, pl.ds(sid * CNT_PART, CNT_PART)])


def _count_edges(dst2d, et2d):
    k = pl.kernel(
        _count_body,
        out_type=jax.ShapeDtypeStruct((NC, CROWS, 128), jnp.float32),
        mesh=_mesh,
        scratch_types=[
            pltpu.VMEM_SHARED((CROWS, 128), jnp.float32),
            pltpu.VMEM((GB, 128), jnp.int32),
            pltpu.VMEM((GB, 128), jnp.int32),
            pltpu.VMEM((GB * 4, OH), jnp.int32),
            pltpu.VMEM((GB, 128), jnp.int32),
            pltpu.VMEM((GB, 128), jnp.int32),
            pltpu.VMEM((OH, 128), jnp.float32),
        ],
        compiler_params=_sc_params,
    )
    return k(dst2d, et2d)


def _scale_body(cnt_ref, scale_ref):
    c = cnt_ref[0] + cnt_ref[1]
    scale_ref[...] = 1.0 / jnp.clip(c, 1.0, None)


def _build_scale(cnt):
    b = 8
    return pl.pallas_call(
        _scale_body,
        grid=(CROWS // b,),
        in_specs=[pl.BlockSpec((NC, b, 128), lambda i: (0, i, 0))],
        out_specs=pl.BlockSpec((b, 128), lambda i: (i, 0)),
        out_shape=jax.ShapeDtypeStruct((CROWS, 128), jnp.float32),
    )(cnt)


def _bcast_lane(v, i):
    idx = jnp.full((LANES,), i, jnp.int32)
    return lax.gather(
        v, idx[:, None],
        lax.GatherDimensionNumbers(offset_dims=(), collapsed_slice_dims=(0,),
                                   start_index_map=(0,)),
        slice_sizes=(1,), mode=lax.GatherScatterMode.PROMISE_IN_BOUNDS)


def _agg_body(src_hbm, et_hbm, dst_hbm, y_hbm, scale_hbm, out_hbm,
              acc_sh, scale_sh, rows, srows, srcv, etv, dstv, gidxv,
              srowv, colv):
    cidx = lax.axis_index("c")
    sid = lax.axis_index("s")
    wid = sid * NC + cidx
    it = lax.iota(jnp.int32, LANES)

    @pl.loop(0, 128)
    def _z(i):
        rk = rows.at[i]
        for l in range(8):
            rk[pl.ds(l * LANES, LANES)] = jnp.zeros((LANES,), jnp.float32)

    abase = sid * ACC_PART

    @pl.loop(0, ACC_PART // 128)
    def _za(i):
        pltpu.sync_copy(rows, acc_sh.at[pl.ds(abase + i * 128, 128)])

    pltpu.sync_copy(rows.at[pl.ds(0, ACC_PART % 128)],
                    acc_sh.at[pl.ds(abase + (ACC_PART // 128) * 128,
                                    ACC_PART % 128)])

    pltpu.sync_copy(scale_hbm.at[pl.ds(sid * CNT_PART, CNT_PART)],
                    scale_sh.at[pl.ds(sid * CNT_PART, CNT_PART)])

    plsc.subcore_barrier()

    @pl.loop(0, NGROUPS)
    def _grp(g):
        rb = wid * WROWS + g * GB
        pltpu.sync_copy(src_hbm.at[pl.ds(rb, GB)], srcv)
        pltpu.sync_copy(et_hbm.at[pl.ds(rb, GB)], etv)
        pltpu.sync_copy(dst_hbm.at[pl.ds(rb, GB)], dstv)

        @pl.loop(0, GB)
        def _idx(j):
            rsrc, ret, rdst = srcv.at[j], etv.at[j], dstv.at[j]
            rgi, rsr, rco = gidxv.at[j], srowv.at[j], colv.at[j]
            for l in range(8):
                sl = pl.ds(l * LANES, LANES)
                et16 = ret[sl]
                rgi[sl] = et16 * N + rsrc[sl]
                seg16 = rdst[sl] * R + et16
                rsr[sl] = jax.lax.shift_right_logical(seg16, 7)
                rco[sl] = jax.lax.bitwise_and(seg16, 127)

        @pl.loop(0, GB)
        def _pay(j):
            pltpu.sync_copy(y_hbm.at[gidxv.at[j]], rows)
            rsr, rco = srowv.at[j], colv.at[j]

            @pl.loop(0, 8)
            def _sub(s):
                pltpu.sync_copy(
                    scale_sh.at[rsr[pl.ds(s * LANES, LANES)]], srows)
                cv = rco[pl.ds(s * LANES, LANES)]
                sv = plsc.load_gather(srows, [it, cv])

                @pl.loop(0, LANES)
                def _edge(i):
                    b = _bcast_lane(sv, i)
                    rk = rows.at[s * LANES + i]
                    for l in range(8):
                        sl = pl.ds(l * LANES, LANES)
                        rk[sl] = rk[sl] * b

            pltpu.sync_copy(rows, acc_sh.at[dstv.at[j]], add=True)

    plsc.subcore_barrier()
    pltpu.sync_copy(acc_sh.at[pl.ds(sid * ACC_PART, ACC_PART)],
                    out_hbm.at[cidx, pl.ds(sid * ACC_PART, ACC_PART)])


def _aggregate(src2d, et2d, dst2d, y, scale):
    k = pl.kernel(
        _agg_body,
        out_type=jax.ShapeDtypeStruct((NC, NP, D), jnp.float32),
        mesh=_mesh,
        scratch_types=[
            pltpu.VMEM_SHARED((NP, D), jnp.float32),
            pltpu.VMEM_SHARED((CROWS, 128), jnp.float32),
            pltpu.VMEM((128, D), jnp.float32),
            pltpu.VMEM((LANES, 128), jnp.float32),
            pltpu.VMEM((GB, 128), jnp.int32),
            pltpu.VMEM((GB, 128), jnp.int32),
            pltpu.VMEM((GB, 128), jnp.int32),
            pltpu.VMEM((GB, 128), jnp.int32),
            pltpu.VMEM((GB, 128), jnp.int32),
            pltpu.VMEM((GB, 128), jnp.int32),
        ],
        compiler_params=_sc_params,
    )
    return k(src2d, et2d, dst2d, y, scale)


def _final_body(acc_ref, x_ref, root_ref, bias_ref, out_ref):
    out_ref[...] = (acc_ref[0] + acc_ref[1]
                    + jnp.dot(x_ref[...], root_ref[...],
                              preferred_element_type=jnp.float32)
                    + bias_ref[...])


def _finalize(acc, x, root, bias):
    b = 1000
    return pl.pallas_call(
        _final_body,
        grid=(N // b,),
        in_specs=[
            pl.BlockSpec((NC, b, D), lambda i: (0, i, 0)),
            pl.BlockSpec((b, D), lambda i: (i, 0)),
            pl.BlockSpec((D, D), lambda i: (0, 0)),
            pl.BlockSpec((1, D), lambda i: (0, 0)),
        ],
        out_specs=pl.BlockSpec((b, D), lambda i: (i, 0)),
        out_shape=jax.ShapeDtypeStruct((N, D), jnp.float32),
    )(acc[:, :N], x, root, bias.reshape(1, D))


def kernel(x, edge_index, edge_type, weight, root, bias):
    src = edge_index[0].astype(jnp.int32)
    dst = edge_index[1].astype(jnp.int32)
    et = edge_type.astype(jnp.int32)

    npad = EP - E
    src2d = jnp.concatenate(
        [src, jnp.zeros((npad,), jnp.int32)]).reshape(ROWS, 128)
    dst2d = jnp.concatenate(
        [dst, jnp.full((npad,), N, jnp.int32)]).reshape(ROWS, 128)
    et2d = jnp.concatenate(
        [et, jnp.zeros((npad,), jnp.int32)]).reshape(ROWS, 128)

    y = _build_y(x, weight)
    cnt = _count_edges(dst2d, et2d)
    scale = _build_scale(cnt)
    acc = _aggregate(src2d, et2d, dst2d, y, scale)
    return _finalize(acc, x, root, bias)


# speedup vs baseline: 3.1646x; 1.0560x over previous
import dataclasses
import functools

import jax
import jax.numpy as jnp
from jax import lax
from jax.experimental import pallas as pl
from jax.experimental.pallas import tpu as pltpu
from jax.experimental.pallas import tpu_sc as plsc

N = 10000
E = 320000
D = 128
R = 8

NC = 2
NS = 16
NW = NC * NS
LANES = 16

WROWS = 80
ROWS = NW * WROWS
EP = ROWS * 128
GB = 8
NGROUPS = WROWS // GB

SEGP = 81920
CROWS = SEGP // 128
CNT_PART = CROWS // NS
NP = 10112
ACC_PART = NP // NS
OH = 32

_mesh = plsc.VectorSubcoreMesh(core_axis_name="c", subcore_axis_name="s",
                               num_cores=NC, num_subcores=NS)

_sc_params = pltpu.CompilerParams()
if "needs_layout_passes" in pltpu.CompilerParams.__dataclass_fields__:
    _sc_params = dataclasses.replace(_sc_params, needs_layout_passes=False)


def _y_body(x_ref, w_ref, y_ref):
    y_ref[...] = jnp.dot(x_ref[...], w_ref[0],
                         preferred_element_type=jnp.float32)


def _build_y(x, weight):
    bn = 1000
    return pl.pallas_call(
        _y_body,
        grid=(R, N // bn),
        in_specs=[
            pl.BlockSpec((bn, D), lambda r, i: (i, 0)),
            pl.BlockSpec((1, D, D), lambda r, i: (r, 0, 0)),
        ],
        out_specs=pl.BlockSpec((bn, D), lambda r, i: (r * (N // bn) + i, 0)),
        out_shape=jax.ShapeDtypeStruct((R * N, D), jnp.float32),
    )(x, weight)


def _count_body(dst_hbm, et_hbm, cnt_hbm, cnt_sh, dstv, etv, rowv, grpv,
                lanev, onehot):
    cidx = lax.axis_index("c")
    sid = lax.axis_index("s")
    wid = sid * NC + cidx
    it = lax.iota(jnp.int32, LANES)

    @pl.loop(0, OH)
    def _z(i):
        rk = onehot.at[i]
        for l in range(8):
            rk[pl.ds(l * LANES, LANES)] = jnp.zeros((LANES,), jnp.float32)

    zbase = sid * CNT_PART
    pltpu.sync_copy(onehot, cnt_sh.at[pl.ds(zbase, OH)])
    pltpu.sync_copy(onehot.at[pl.ds(0, CNT_PART - OH)],
                    cnt_sh.at[pl.ds(zbase + OH, CNT_PART - OH)])

    plsc.subcore_barrier()

    @pl.loop(0, NGROUPS)
    def _grp(g):
        rb = wid * WROWS + g * GB
        pltpu.sync_copy(dst_hbm.at[pl.ds(rb, GB)], dstv)
        pltpu.sync_copy(et_hbm.at[pl.ds(rb, GB)], etv)

        @pl.loop(0, GB)
        def _idx(j):
            rd, re = dstv.at[j], etv.at[j]
            rg, rl = grpv.at[j], lanev.at[j]
            for l in range(8):
                sl = pl.ds(l * LANES, LANES)
                seg16 = rd[sl] * R + re[sl]
                rowv.at[j * 4 + l // 2][pl.ds((l % 2) * LANES, LANES)] = (
                    jax.lax.shift_right_logical(seg16, 7))
                rg[sl] = jax.lax.bitwise_and(
                    jax.lax.shift_right_logical(seg16, 4), 7)
                rl[sl] = jax.lax.bitwise_and(seg16, 15)

        @pl.loop(0, GB * 4)
        def _sub(q):
            j = lax.div(q, 4)
            base = lax.rem(q, 4) * OH
            rg, rl = grpv.at[j], lanev.at[j]
            gvec = [rg[pl.ds(base, LANES)], rg[pl.ds(base + LANES, LANES)]]
            lvec = [rl[pl.ds(base, LANES)], rl[pl.ds(base + LANES, LANES)]]
            gs = []
            for i in range(OH):
                gi = gvec[i // LANES][i % LANES]
                li = lvec[i // LANES][i % LANES]
                gs.append(gi)
                onehot.at[i][pl.ds(gi * LANES, LANES)] = jnp.where(
                    it == li, 1.0, 0.0).astype(jnp.float32)
            pltpu.sync_copy(onehot, cnt_sh.at[rowv.at[q]], add=True)
            for i in range(OH):
                onehot.at[i][pl.ds(gs[i] * LANES, LANES)] = jnp.zeros(
                    (LANES,), jnp.float32)

    plsc.subcore_barrier()
    pltpu.sync_copy(cnt_sh.at[pl.ds(sid * CNT_PART, CNT_PART)],
                    cnt_hbm.at[cidx, pl.ds(sid * CNT_PART, CNT_PART)])


def _count_edges(dst2d, et2d):
    k = pl.kernel(
        _count_body,
        out_type=jax.ShapeDtypeStruct((NC, CROWS, 128), jnp.float32),
        mesh=_mesh,
        scratch_types=[
            pltpu.VMEM_SHARED((CROWS, 128), jnp.float32),
            pltpu.VMEM((GB, 128), jnp.int32),
            pltpu.VMEM((GB, 128), jnp.int32),
            pltpu.VMEM((GB * 4, OH), jnp.int32),
            pltpu.VMEM((GB, 128), jnp.int32),
            pltpu.VMEM((GB, 128), jnp.int32),
            pltpu.VMEM((OH, 128), jnp.float32),
        ],
        compiler_params=_sc_params,
    )
    return k(dst2d, et2d)


def _scale_body(cnt_ref, scale_ref):
    c = cnt_ref[0] + cnt_ref[1]
    scale_ref[...] = 1.0 / jnp.clip(c, 1.0, None)


def _build_scale(cnt):
    b = 8
    return pl.pallas_call(
        _scale_body,
        grid=(CROWS // b,),
        in_specs=[pl.BlockSpec((NC, b, 128), lambda i: (0, i, 0))],
        out_specs=pl.BlockSpec((b, 128), lambda i: (i, 0)),
        out_shape=jax.ShapeDtypeStruct((CROWS, 128), jnp.float32),
    )(cnt)


def _bcast_lane(v, i):
    idx = jnp.full((LANES,), i, jnp.int32)
    return lax.gather(
        v, idx[:, None],
        lax.GatherDimensionNumbers(offset_dims=(), collapsed_slice_dims=(0,),
                                   start_index_map=(0,)),
        slice_sizes=(1,), mode=lax.GatherScatterMode.PROMISE_IN_BOUNDS)


def _agg_body(src_hbm, et_hbm, dst_hbm, y_hbm, scale_hbm, out_hbm,
              acc_sh, scale_sh, rows, srows, srcv, etv, dstv, gidxv,
              srowv, colv):
    cidx = lax.axis_index("c")
    sid = lax.axis_index("s")
    wid = sid * NC + cidx
    it = lax.iota(jnp.int32, LANES)

    @pl.loop(0, 128)
    def _z(i):
        rk = rows.at[i]
        for l in range(8):
            rk[pl.ds(l * LANES, LANES)] = jnp.zeros((LANES,), jnp.float32)

    abase = sid * ACC_PART

    @pl.loop(0, ACC_PART // 128)
    def _za(i):
        pltpu.sync_copy(rows, acc_sh.at[pl.ds(abase + i * 128, 128)])

    pltpu.sync_copy(rows.at[pl.ds(0, ACC_PART % 128)],
                    acc_sh.at[pl.ds(abase + (ACC_PART // 128) * 128,
                                    ACC_PART % 128)])

    pltpu.sync_copy(scale_hbm.at[pl.ds(sid * CNT_PART, CNT_PART)],
                    scale_sh.at[pl.ds(sid * CNT_PART, CNT_PART)])

    plsc.subcore_barrier()

    @pl.loop(0, NGROUPS)
    def _grp(g):
        rb = wid * WROWS + g * GB
        pltpu.sync_copy(src_hbm.at[pl.ds(rb, GB)], srcv)
        pltpu.sync_copy(et_hbm.at[pl.ds(rb, GB)], etv)
        pltpu.sync_copy(dst_hbm.at[pl.ds(rb, GB)], dstv)

        @pl.loop(0, GB)
        def _idx(j):
            rsrc, ret, rdst = srcv.at[j], etv.at[j], dstv.at[j]
            rgi, rsr, rco = gidxv.at[j], srowv.at[j], colv.at[j]
            for l in range(8):
                sl = pl.ds(l * LANES, LANES)
                et16 = ret[sl]
                rgi[sl] = et16 * N + rsrc[sl]
                seg16 = rdst[sl] * R + et16
                rsr[sl] = jax.lax.shift_right_logical(seg16, 7)
                rco[sl] = jax.lax.bitwise_and(seg16, 127)

        @pl.loop(0, GB)
        def _pay(j):
            pltpu.sync_copy(y_hbm.at[gidxv.at[j]], rows)
            pltpu.sync_copy(scale_sh.at[srowv.at[j]], srows)
            rco = colv.at[j]

            @pl.loop(0, 8)
            def _sub(s):
                cv = rco[pl.ds(s * LANES, LANES)]
                sv = plsc.load_gather(srows, [s * LANES + it, cv])

                @pl.loop(0, LANES)
                def _edge(i):
                    b = _bcast_lane(sv, i)
                    rk = rows.at[s * LANES + i]
                    for l in range(8):
                        sl = pl.ds(l * LANES, LANES)
                        rk[sl] = rk[sl] * b

            pltpu.sync_copy(rows, acc_sh.at[dstv.at[j]], add=True)

    plsc.subcore_barrier()
    pltpu.sync_copy(acc_sh.at[pl.ds(sid * ACC_PART, ACC_PART)],
                    out_hbm.at[cidx, pl.ds(sid * ACC_PART, ACC_PART)])


def _aggregate(src2d, et2d, dst2d, y, scale):
    k = pl.kernel(
        _agg_body,
        out_type=jax.ShapeDtypeStruct((NC, NP, D), jnp.float32),
        mesh=_mesh,
        scratch_types=[
            pltpu.VMEM_SHARED((NP, D), jnp.float32),
            pltpu.VMEM_SHARED((CROWS, 128), jnp.float32),
            pltpu.VMEM((128, D), jnp.float32),
            pltpu.VMEM((128, 128), jnp.float32),
            pltpu.VMEM((GB, 128), jnp.int32),
            pltpu.VMEM((GB, 128), jnp.int32),
            pltpu.VMEM((GB, 128), jnp.int32),
            pltpu.VMEM((GB, 128), jnp.int32),
            pltpu.VMEM((GB, 128), jnp.int32),
            pltpu.VMEM((GB, 128), jnp.int32),
        ],
        compiler_params=_sc_params,
    )
    return k(src2d, et2d, dst2d, y, scale)


def _final_body(acc_ref, x_ref, root_ref, bias_ref, out_ref):
    out_ref[...] = (acc_ref[0] + acc_ref[1]
                    + jnp.dot(x_ref[...], root_ref[...],
                              preferred_element_type=jnp.float32)
                    + bias_ref[...])


def _finalize(acc, x, root, bias):
    b = 1000
    return pl.pallas_call(
        _final_body,
        grid=(N // b,),
        in_specs=[
            pl.BlockSpec((NC, b, D), lambda i: (0, i, 0)),
            pl.BlockSpec((b, D), lambda i: (i, 0)),
            pl.BlockSpec((D, D), lambda i: (0, 0)),
            pl.BlockSpec((1, D), lambda i: (0, 0)),
        ],
        out_specs=pl.BlockSpec((b, D), lambda i: (i, 0)),
        out_shape=jax.ShapeDtypeStruct((N, D), jnp.float32),
    )(acc[:, :N], x, root, bias.reshape(1, D))


def kernel(x, edge_index, edge_type, weight, root, bias):
    src = edge_index[0].astype(jnp.int32)
    dst = edge_index[1].astype(jnp.int32)
    et = edge_type.astype(jnp.int32)

    npad = EP - E
    src2d = jnp.concatenate(
        [src, jnp.zeros((npad,), jnp.int32)]).reshape(ROWS, 128)
    dst2d = jnp.concatenate(
        [dst, jnp.full((npad,), N, jnp.int32)]).reshape(ROWS, 128)
    et2d = jnp.concatenate(
        [et, jnp.zeros((npad,), jnp.int32)]).reshape(ROWS, 128)

    y = _build_y(x, weight)
    cnt = _count_edges(dst2d, et2d)
    scale = _build_scale(cnt)
    acc = _aggregate(src2d, et2d, dst2d, y, scale)
    return _finalize(acc, x, root, bias)


# speedup vs baseline: 3.2105x; 1.0145x over previous
import dataclasses
import functools

import jax
import jax.numpy as jnp
from jax import lax
from jax.experimental import pallas as pl
from jax.experimental.pallas import tpu as pltpu
from jax.experimental.pallas import tpu_sc as plsc

N = 10000
E = 320000
D = 128
R = 8

NC = 2
NS = 16
NW = NC * NS
LANES = 16

WROWS = 80
ROWS = NW * WROWS
EP = ROWS * 128
GB = 16
NGROUPS = WROWS // GB

SEGP = 81920
CROWS = SEGP // 128
CNT_PART = CROWS // NS
NP = 10112
ACC_PART = NP // NS
OH = 32

_mesh = plsc.VectorSubcoreMesh(core_axis_name="c", subcore_axis_name="s",
                               num_cores=NC, num_subcores=NS)

_sc_params = pltpu.CompilerParams()
if "needs_layout_passes" in pltpu.CompilerParams.__dataclass_fields__:
    _sc_params = dataclasses.replace(_sc_params, needs_layout_passes=False)


def _y_body(x_ref, w_ref, y_ref):
    y_ref[...] = jnp.dot(x_ref[...], w_ref[0],
                         preferred_element_type=jnp.float32)


def _build_y(x, weight):
    bn = 1000
    return pl.pallas_call(
        _y_body,
        grid=(R, N // bn),
        in_specs=[
            pl.BlockSpec((bn, D), lambda r, i: (i, 0)),
            pl.BlockSpec((1, D, D), lambda r, i: (r, 0, 0)),
        ],
        out_specs=pl.BlockSpec((bn, D), lambda r, i: (r * (N // bn) + i, 0)),
        out_shape=jax.ShapeDtypeStruct((R * N, D), jnp.float32),
    )(x, weight)


def _count_body(dst_hbm, et_hbm, cnt_hbm, cnt_sh, dstv, etv, rowv, grpv,
                lanev, onehot):
    cidx = lax.axis_index("c")
    sid = lax.axis_index("s")
    wid = sid * NC + cidx
    it = lax.iota(jnp.int32, LANES)

    @pl.loop(0, OH)
    def _z(i):
        rk = onehot.at[i]
        for l in range(8):
            rk[pl.ds(l * LANES, LANES)] = jnp.zeros((LANES,), jnp.float32)

    zbase = sid * CNT_PART
    pltpu.sync_copy(onehot, cnt_sh.at[pl.ds(zbase, OH)])
    pltpu.sync_copy(onehot.at[pl.ds(0, CNT_PART - OH)],
                    cnt_sh.at[pl.ds(zbase + OH, CNT_PART - OH)])

    plsc.subcore_barrier()

    @pl.loop(0, NGROUPS)
    def _grp(g):
        rb = wid * WROWS + g * GB
        pltpu.sync_copy(dst_hbm.at[pl.ds(rb, GB)], dstv)
        pltpu.sync_copy(et_hbm.at[pl.ds(rb, GB)], etv)

        @pl.loop(0, GB)
        def _idx(j):
            rd, re = dstv.at[j], etv.at[j]
            rg, rl = grpv.at[j], lanev.at[j]
            for l in range(8):
                sl = pl.ds(l * LANES, LANES)
                seg16 = rd[sl] * R + re[sl]
                rowv.at[j * 4 + l // 2][pl.ds((l % 2) * LANES, LANES)] = (
                    jax.lax.shift_right_logical(seg16, 7))
                rg[sl] = jax.lax.bitwise_and(
                    jax.lax.shift_right_logical(seg16, 4), 7)
                rl[sl] = jax.lax.bitwise_and(seg16, 15)

        @pl.loop(0, GB * 4)
        def _sub(q):
            j = lax.div(q, 4)
            base = lax.rem(q, 4) * OH
            rg, rl = grpv.at[j], lanev.at[j]
            gvec = [rg[pl.ds(base, LANES)], rg[pl.ds(base + LANES, LANES)]]
            lvec = [rl[pl.ds(base, LANES)], rl[pl.ds(base + LANES, LANES)]]
            gs = []
            for i in range(OH):
                gi = gvec[i // LANES][i % LANES]
                li = lvec[i // LANES][i % LANES]
                gs.append(gi)
                onehot.at[i][pl.ds(gi * LANES, LANES)] = jnp.where(
                    it == li, 1.0, 0.0).astype(jnp.float32)
            pltpu.sync_copy(onehot, cnt_sh.at[rowv.at[q]], add=True)
            for i in range(OH):
                onehot.at[i][pl.ds(gs[i] * LANES, LANES)] = jnp.zeros(
                    (LANES,), jnp.float32)

    plsc.subcore_barrier()
    pltpu.sync_copy(cnt_sh.at[pl.ds(sid * CNT_PART, CNT_PART)],
                    cnt_hbm.at[cidx, pl.ds(sid * CNT_PART, CNT_PART)])


def _count_edges(dst2d, et2d):
    k = pl.kernel(
        _count_body,
        out_type=jax.ShapeDtypeStruct((NC, CROWS, 128), jnp.float32),
        mesh=_mesh,
        scratch_types=[
            pltpu.VMEM_SHARED((CROWS, 128), jnp.float32),
            pltpu.VMEM((GB, 128), jnp.int32),
            pltpu.VMEM((GB, 128), jnp.int32),
            pltpu.VMEM((GB * 4, OH), jnp.int32),
            pltpu.VMEM((GB, 128), jnp.int32),
            pltpu.VMEM((GB, 128), jnp.int32),
            pltpu.VMEM((OH, 128), jnp.float32),
        ],
        compiler_params=_sc_params,
    )
    return k(dst2d, et2d)


def _scale_body(cnt_ref, scale_ref):
    c = cnt_ref[0] + cnt_ref[1]
    scale_ref[...] = 1.0 / jnp.clip(c, 1.0, None)


def _build_scale(cnt):
    b = 8
    return pl.pallas_call(
        _scale_body,
        grid=(CROWS // b,),
        in_specs=[pl.BlockSpec((NC, b, 128), lambda i: (0, i, 0))],
        out_specs=pl.BlockSpec((b, 128), lambda i: (i, 0)),
        out_shape=jax.ShapeDtypeStruct((CROWS, 128), jnp.float32),
    )(cnt)


def _bcast_lane(v, i):
    idx = jnp.full((LANES,), i, jnp.int32)
    return lax.gather(
        v, idx[:, None],
        lax.GatherDimensionNumbers(offset_dims=(), collapsed_slice_dims=(0,),
                                   start_index_map=(0,)),
        slice_sizes=(1,), mode=lax.GatherScatterMode.PROMISE_IN_BOUNDS)


def _agg_body(src_hbm, et_hbm, dst_hbm, y_hbm, scale_hbm, out_hbm,
              acc_sh, scale_sh, rows, srows, srcv, etv, dstv, gidxv,
              srowv, colv):
    cidx = lax.axis_index("c")
    sid = lax.axis_index("s")
    wid = sid * NC + cidx
    it = lax.iota(jnp.int32, LANES)

    @pl.loop(0, 128)
    def _z(i):
        rk = rows.at[i]
        for l in range(8):
            rk[pl.ds(l * LANES, LANES)] = jnp.zeros((LANES,), jnp.float32)

    abase = sid * ACC_PART

    @pl.loop(0, ACC_PART // 128)
    def _za(i):
        pltpu.sync_copy(rows, acc_sh.at[pl.ds(abase + i * 128, 128)])

    pltpu.sync_copy(rows.at[pl.ds(0, ACC_PART % 128)],
                    acc_sh.at[pl.ds(abase + (ACC_PART // 128) * 128,
                                    ACC_PART % 128)])

    pltpu.sync_copy(scale_hbm.at[pl.ds(sid * CNT_PART, CNT_PART)],
                    scale_sh.at[pl.ds(sid * CNT_PART, CNT_PART)])

    plsc.subcore_barrier()

    @pl.loop(0, NGROUPS)
    def _grp(g):
        rb = wid * WROWS + g * GB
        pltpu.sync_copy(src_hbm.at[pl.ds(rb, GB)], srcv)
        pltpu.sync_copy(et_hbm.at[pl.ds(rb, GB)], etv)
        pltpu.sync_copy(dst_hbm.at[pl.ds(rb, GB)], dstv)

        @pl.loop(0, GB)
        def _idx(j):
            rsrc, ret, rdst = srcv.at[j], etv.at[j], dstv.at[j]
            rgi, rsr, rco = gidxv.at[j], srowv.at[j], colv.at[j]
            for l in range(8):
                sl = pl.ds(l * LANES, LANES)
                et16 = ret[sl]
                rgi[sl] = et16 * N + rsrc[sl]
                seg16 = rdst[sl] * R + et16
                rsr[sl] = jax.lax.shift_right_logical(seg16, 7)
                rco[sl] = jax.lax.bitwise_and(seg16, 127)

        @pl.loop(0, GB)
        def _pay(j):
            pltpu.sync_copy(y_hbm.at[gidxv.at[j]], rows)
            pltpu.sync_copy(scale_sh.at[srowv.at[j]], srows)
            rco = colv.at[j]

            @pl.loop(0, 8)
            def _sub(s):
                cv = rco[pl.ds(s * LANES, LANES)]
                sv = plsc.load_gather(srows, [s * LANES + it, cv])

                @pl.loop(0, LANES)
                def _edge(i):
                    b = _bcast_lane(sv, i)
                    rk = rows.at[s * LANES + i]
                    for l in range(8):
                        sl = pl.ds(l * LANES, LANES)
                        rk[sl] = rk[sl] * b

            pltpu.sync_copy(rows, acc_sh.at[dstv.at[j]], add=True)

    plsc.subcore_barrier()
    pltpu.sync_copy(acc_sh.at[pl.ds(sid * ACC_PART, ACC_PART)],
                    out_hbm.at[cidx, pl.ds(sid * ACC_PART, ACC_PART)])


def _aggregate(src2d, et2d, dst2d, y, scale):
    k = pl.kernel(
        _agg_body,
        out_type=jax.ShapeDtypeStruct((NC, NP, D), jnp.float32),
        mesh=_mesh,
        scratch_types=[
            pltpu.VMEM_SHARED((NP, D), jnp.float32),
            pltpu.VMEM_SHARED((CROWS, 128), jnp.float32),
            pltpu.VMEM((128, D), jnp.float32),
            pltpu.VMEM((128, 128), jnp.float32),
            pltpu.VMEM((GB, 128), jnp.int32),
            pltpu.VMEM((GB, 128), jnp.int32),
            pltpu.VMEM((GB, 128), jnp.int32),
            pltpu.VMEM((GB, 128), jnp.int32),
            pltpu.VMEM((GB, 128), jnp.int32),
            pltpu.VMEM((GB, 128), jnp.int32),
        ],
        compiler_params=_sc_params,
    )
    return k(src2d, et2d, dst2d, y, scale)


def _final_body(acc_ref, x_ref, root_ref, bias_ref, out_ref):
    out_ref[...] = (acc_ref[0] + acc_ref[1]
                    + jnp.dot(x_ref[...], root_ref[...],
                              preferred_element_type=jnp.float32)
                    + bias_ref[...])


def _finalize(acc, x, root, bias):
    b = 1000
    return pl.pallas_call(
        _final_body,
        grid=(N // b,),
        in_specs=[
            pl.BlockSpec((NC, b, D), lambda i: (0, i, 0)),
            pl.BlockSpec((b, D), lambda i: (i, 0)),
            pl.BlockSpec((D, D), lambda i: (0, 0)),
            pl.BlockSpec((1, D), lambda i: (0, 0)),
        ],
        out_specs=pl.BlockSpec((b, D), lambda i: (i, 0)),
        out_shape=jax.ShapeDtypeStruct((N, D), jnp.float32),
    )(acc[:, :N], x, root, bias.reshape(1, D))


def kernel(x, edge_index, edge_type, weight, root, bias):
    src = edge_index[0].astype(jnp.int32)
    dst = edge_index[1].astype(jnp.int32)
    et = edge_type.astype(jnp.int32)

    npad = EP - E
    src2d = jnp.concatenate(
        [src, jnp.zeros((npad,), jnp.int32)]).reshape(ROWS, 128)
    dst2d = jnp.concatenate(
        [dst, jnp.full((npad,), N, jnp.int32)]).reshape(ROWS, 128)
    et2d = jnp.concatenate(
        [et, jnp.zeros((npad,), jnp.int32)]).reshape(ROWS, 128)

    y = _build_y(x, weight)
    cnt = _count_edges(dst2d, et2d)
    scale = _build_scale(cnt)
    acc = _aggregate(src2d, et2d, dst2d, y, scale)
    return _finalize(acc, x, root, bias)
